# baseline (speedup 1.0000x reference)
import functools

import jax
import jax.numpy as jnp
import numpy as np
from jax import lax
from jax.experimental import pallas as pl
from jax.experimental.pallas import tpu as pltpu
from jax.experimental.pallas import tpu_sc as plsc

PENALTY = 0.99
NEG = float(np.finfo(np.float32).min)


def _rot_half(x, d):
    h = d // 2
    return jnp.concatenate([-x[:, h:], x[:, :h]], axis=1)


def _proj_body(h_ref, wq_ref, wk_ref, wv_ref, q_ref, k_ref, v_ref):
    h = h_ref[...]
    dn = (((1,), (1,)), ((), ()))
    q_ref[0] = jax.lax.dot_general(h, wq_ref[0], dn, preferred_element_type=jnp.float32)
    k_ref[0] = jax.lax.dot_general(h, wk_ref[0], dn, preferred_element_type=jnp.float32)
    v_ref[0] = jax.lax.dot_general(h, wv_ref[0], dn, preferred_element_type=jnp.float32)


def _scores_body(q_ref, k_ref, cq_ref, sq_ref, ck_ref, sk_ref, a_ref, *, d, scale):
    q = q_ref[0]
    k = k_ref[0]
    qr = q * cq_ref[...] + _rot_half(q, d) * sq_ref[...]
    kr = k * ck_ref[...] + _rot_half(k, d) * sk_ref[...]
    dn = (((1,), (1,)), ((), ()))
    a_ref[0] = jax.lax.dot_general(qr, kr, dn, preferred_element_type=jnp.float32) * scale


def _evict_body(a_ref, sc_in_ref, e_in_ref, e_ref, score_ref, *,
                s, h, rb, w, w_prev, t0, recent, cache, do_evict):
    tb = pl.program_id(0)
    col = jax.lax.broadcasted_iota(jnp.int32, (h, w), 1)

    @pl.when(tb == 0)
    def _init():
        score_ref[...] = jnp.where(col >= w_prev, 0.0, sc_in_ref[...])
        e_ref[...] = jnp.where(col >= w_prev, s + 1, e_in_ref[...])

    for i in range(rb):
        t = t0 + tb * rb + i
        e = e_ref[...]
        score = score_ref[...]
        active = (col <= t) & (e > t)
        ex = jnp.exp(jnp.where(active, a_ref[:, i, :], NEG))
        z = jnp.sum(ex, axis=1, keepdims=True)
        score = jnp.where(active, PENALTY * score + ex / z, 0.0)
        if do_evict:
            cand = (e > t) & (col <= t - recent)
            sc = jnp.where(cand, score, jnp.inf)
            mn = jnp.min(sc, axis=1, keepdims=True)
            evict = jnp.max(jnp.where(cand & (sc == mn), col, -1),
                            axis=1, keepdims=True)
            do = jnp.logical_and(t >= cache, t < s - 1)
            e_ref[...] = jnp.where(jnp.logical_and(do, col == evict), t + 1, e)
        score_ref[...] = score


def _perm(v, idx):
    dn = lax.GatherDimensionNumbers(offset_dims=(), collapsed_slice_dims=(0,),
                                    start_index_map=(0,))
    return lax.gather(v, idx[:, None], dn, (1,),
                      mode=lax.GatherScatterMode.PROMISE_IN_BOUNDS)


def _allred(v, op, lane):
    for k in (8, 4, 2, 1):
        v = op(v, _perm(v, lax.rem(lane + k, 16)))
    return v


def _sc_evict_body(af_hbm, sf_hbm, e_hbm, wrow, score, eloc,
                   gi0, gb0, wsem, gsem, *,
                   s, nheads, heavy, recent, cache, nc):
    f32 = jnp.float32
    i32 = jnp.int32
    NBUF = 4
    PREF = 2
    NH = (heavy + 15) // 16
    NR = (recent + 16) // 16
    wid = lax.axis_index("s") * nc + lax.axis_index("c")
    h = wid

    @pl.when(h < nheads)
    def _body():
        lane = lax.broadcasted_iota(i32, (16,), 0)
        zi = jnp.zeros((16,), i32)
        zf = jnp.zeros((16,), f32)
        INF = jnp.full((16,), np.inf, f32)
        NEG1 = jnp.full((16,), -1, i32)

        SL = 512
        pltpu.sync_copy(sf_hbm.at[pl.ds(h * s, SL)], score.at[pl.ds(0, SL)])

        def init_s(j, c):
            score[pl.ds(SL + 16 * j, 16)] = zf
            return c
        lax.fori_loop(0, (s + 16 - SL) // 16, init_s, 0)

        def init_e(j, c):
            eloc[pl.ds(16 * j, 16)] = zi + (s + 1)
            return c
        lax.fori_loop(0, s // 16, init_e, 0)

        def row_off(t):
            return (h * s + t) * s

        WL = 224

        def wst_of(t):
            return jnp.minimum(((t - recent - 8) // 8) * 8, s - WL)

        def wdma_in(t):
            b = lax.rem(t, NBUF)
            pltpu.make_async_copy(af_hbm.at[pl.ds(row_off(t) + wst_of(t), WL)],
                                  wrow.at[pl.ds(b * 256, WL)], wsem.at[b]).start()

        def wwait_in(t):
            b = lax.rem(t, NBUF)
            pltpu.make_async_copy(af_hbm.at[pl.ds(row_off(t) + wst_of(t), WL)],
                                  wrow.at[pl.ds(b * 256, WL)], wsem.at[b]).wait()

        def gissue(t1, hidx):
            gbase = (h * s + t1) * s
            for j in range(NH):
                gi0[pl.ds(16 * j, 16)] = gbase + hidx[j]
            pltpu.make_async_copy(af_hbm.at[gi0], gb0, gsem.at[0]).start()

        def gwait():
            pltpu.make_async_copy(af_hbm.at[gi0], gb0, gsem.at[0]).wait()

        hidx0 = tuple(16 * j + lane for j in range(NH))
        hsc0 = tuple(score[pl.ds(16 * j, 16)] for j in range(NH))
        wdma_in(cache)
        wdma_in(cache + 1)
        gissue(cache, hidx0)

        def step(t, carry):
            hidx, hsc = carry
            wwait_in(t)
            gwait()
            b = lax.rem(t, NBUF)

            @pl.when(t + PREF < s - 1)
            def _():
                wdma_in(t + PREF)

            @pl.when(t + 1 < s - 1)
            def _():
                gissue(t + 1, hidx)

            base = t - recent
            wst = wst_of(t)
            off = base - wst
            exr = []
            zv = zf
            for j in range(NR):
                a = wrow[pl.ds(b * 256 + off + 16 * j, 16)]
                if 16 * (j + 1) <= recent + 1:
                    ex = jnp.exp(a)
                else:
                    ex = jnp.where(16 * j + lane < recent + 1, jnp.exp(a), zf)
                exr.append(ex)
                zv = zv + ex
            pg = base - 1 - wst
            pgal = (pg // 16) * 16
            lv = wrow[pl.ds(b * 256 + pgal, 16)]
            patch = _perm(lv, zi + (pg - pgal))
            gprev = zi + (base - 1)
            exh = []
            for j in range(NH):
                ah = gb0[pl.ds(16 * j, 16)]
                ah = jnp.where(hidx[j] == gprev, patch, ah)
                if 16 * (j + 1) <= heavy:
                    ex = jnp.exp(ah)
                else:
                    ex = jnp.where(16 * j + lane < heavy, jnp.exp(ah), zf)
                exh.append(ex)
                zv = zv + ex
            rz = (zf + 1.0) / _allred(zv, jnp.add, lane)
            gvec = None
            for j in range(NR):
                sl = pl.ds(base + 16 * j, 16)
                old = score[sl]
                if 16 * (j + 1) <= recent + 1:
                    new = PENALTY * old + exr[j] * rz
                else:
                    new = jnp.where(16 * j + lane < recent + 1,
                                    PENALTY * old + exr[j] * rz, old)
                score[sl] = new
                if j == 0:
                    gvec = new
            hsc2 = tuple(PENALTY * hsc[j] + exh[j] * rz for j in range(NH))
            gcand = jnp.where(lane == 0, gvec, INF)
            mv = gcand
            for j in range(NH):
                if 16 * (j + 1) <= heavy:
                    mv = jnp.minimum(mv, hsc2[j])
                else:
                    mv = jnp.minimum(mv, jnp.where(16 * j + lane < heavy,
                                                   hsc2[j], INF))
            mval = _allred(mv, jnp.minimum, lane)
            pv = jnp.where((lane == 0) & (gcand == mval), zi + base, NEG1)
            for j in range(NH):
                if 16 * (j + 1) <= heavy:
                    hit = hsc2[j] == mval
                else:
                    hit = (16 * j + lane < heavy) & (hsc2[j] == mval)
                pv = jnp.maximum(pv, jnp.where(hit, hidx[j], NEG1))
            ev = _allred(pv, jnp.maximum, lane)
            sg = _perm(gvec, lane * 0)
            hidx2 = tuple(jnp.where(hidx[j] == ev, zi + base, hidx[j])
                          for j in range(NH))
            hsc3 = tuple(jnp.where(hidx[j] == ev, sg, hsc2[j])
                         for j in range(NH))
            evs = ev[0]
            al = (evs // 16) * 16
            sl = pl.ds(al, 16)
            eloc[sl] = jnp.where(al + lane == evs, zi + (t + 1), eloc[sl])
            return (hidx2, hsc3)

        lax.fori_loop(cache, s - 1, step, (hidx0, hsc0))
        pltpu.sync_copy(eloc, e_hbm.at[h])


def _attnv_body(a_ref, e_ref, v_ref, o_ref, *, s, h, rb):
    hh = pl.program_id(0)
    rbi = pl.program_id(1)
    a = a_ref[0]
    e_full = e_ref[...]
    hrow = jax.lax.broadcasted_iota(jnp.int32, (h, s), 0)
    e_h = jnp.max(jnp.where(hrow == hh, e_full, 0), axis=0, keepdims=True)
    row = rbi * rb + jax.lax.broadcasted_iota(jnp.int32, (rb, s), 0)
    col = jax.lax.broadcasted_iota(jnp.int32, (rb, s), 1)
    msk = (col <= row) & (row < e_h)
    aa = jnp.where(msk, a, NEG)
    m = jnp.max(aa, axis=1, keepdims=True)
    p = jnp.exp(aa - m)
    p = p / jnp.sum(p, axis=1, keepdims=True)
    dn = (((1,), (0,)), ((), ()))
    o_ref[0] = jax.lax.dot_general(p, v_ref[0], dn, preferred_element_type=jnp.float32)


def _outproj_body(o_ref, wot_ref, y_ref):
    hh = pl.program_id(1)

    @pl.when(hh == 0)
    def _init():
        y_ref[...] = jnp.zeros_like(y_ref)

    dn = (((1,), (0,)), ((), ()))
    y_ref[...] += jax.lax.dot_general(o_ref[0], wot_ref[0], dn, preferred_element_type=jnp.float32)


def _run(hs, Wq, Wk, Wv, Wo, *, s, hid, nheads, d, interpret=False):
    heavy = int(0.1 * s)
    recent = int(0.1 * s)
    cache = heavy + recent
    scale = 1.0 / float(np.sqrt(d).astype(np.float32))
    rb = min(256, s)
    nrb = s // rb
    rb3 = 8
    f32 = jnp.float32

    inv_freq = 1.0 / (10000.0 ** (jnp.arange(0, d, 2, dtype=f32) / d))
    t_ar = jnp.arange(s, dtype=f32)
    freqs = jnp.einsum('i,j->ij', t_ar, inv_freq)
    emb = jnp.concatenate([freqs, freqs], axis=-1)
    cos, sin = jnp.cos(emb), jnp.sin(emb)

    wq3 = Wq.reshape(nheads, d, hid)
    wk3 = Wk.reshape(nheads, d, hid)
    wv3 = Wv.reshape(nheads, d, hid)
    wot3 = Wo.T.reshape(nheads, d, hid)

    q, k, v = pl.pallas_call(
        _proj_body,
        grid=(nheads, nrb),
        in_specs=[
            pl.BlockSpec((rb, hid), lambda hh, i: (i, 0)),
            pl.BlockSpec((1, d, hid), lambda hh, i: (hh, 0, 0)),
            pl.BlockSpec((1, d, hid), lambda hh, i: (hh, 0, 0)),
            pl.BlockSpec((1, d, hid), lambda hh, i: (hh, 0, 0)),
        ],
        out_specs=[
            pl.BlockSpec((1, rb, d), lambda hh, i: (hh, i, 0)),
            pl.BlockSpec((1, rb, d), lambda hh, i: (hh, i, 0)),
            pl.BlockSpec((1, rb, d), lambda hh, i: (hh, i, 0)),
        ],
        out_shape=[jax.ShapeDtypeStruct((nheads, s, d), f32)] * 3,
        interpret=interpret,
    )(hs, wq3, wk3, wv3)

    a = pl.pallas_call(
        functools.partial(_scores_body, d=d, scale=scale),
        grid=(nheads, nrb),
        in_specs=[
            pl.BlockSpec((1, rb, d), lambda hh, i: (hh, i, 0)),
            pl.BlockSpec((1, s, d), lambda hh, i: (hh, 0, 0)),
            pl.BlockSpec((rb, d), lambda hh, i: (i, 0)),
            pl.BlockSpec((rb, d), lambda hh, i: (i, 0)),
            pl.BlockSpec((s, d), lambda hh, i: (0, 0)),
            pl.BlockSpec((s, d), lambda hh, i: (0, 0)),
        ],
        out_specs=pl.BlockSpec((1, rb, s), lambda hh, i: (hh, i, 0)),
        out_shape=jax.ShapeDtypeStruct((nheads, s, s), f32),
        interpret=interpret,
    )(q, k, cos, sin, cos, sin)

    if not interpret and s >= 2048:
        score0 = jnp.zeros((nheads, s), f32)
        e0 = jnp.zeros((nheads, s), jnp.int32)
        cfl = (cache // rb3) * rb3
        _, score_dense = pl.pallas_call(
            functools.partial(_evict_body, s=s, h=nheads, rb=rb3, w=512,
                              w_prev=0, t0=0, recent=recent, cache=cache,
                              do_evict=False),
            grid=(cfl // rb3,),
            in_specs=[
                pl.BlockSpec((nheads, rb3, 512), lambda tb: (0, tb, 0)),
                pl.BlockSpec((nheads, 512), lambda tb: (0, 0)),
                pl.BlockSpec((nheads, 512), lambda tb: (0, 0)),
            ],
            out_specs=[
                pl.BlockSpec((nheads, 512), lambda tb: (0, 0)),
                pl.BlockSpec((nheads, 512), lambda tb: (0, 0)),
            ],
            out_shape=[jax.ShapeDtypeStruct((nheads, s), jnp.int32),
                       jax.ShapeDtypeStruct((nheads, s), f32)],
            interpret=interpret,
        )(a, score0, e0)
        info = plsc.get_sparse_core_info()
        mesh = plsc.VectorSubcoreMesh(core_axis_name="c", subcore_axis_name="s")
        e = pl.kernel(
            functools.partial(_sc_evict_body, s=s, nheads=nheads, heavy=heavy,
                              recent=recent, cache=cache, nc=info.num_cores),
            mesh=mesh,
            out_type=jax.ShapeDtypeStruct((nheads, s), jnp.int32),
            scratch_types=[
                pltpu.VMEM((4 * 256,), f32),
                pltpu.VMEM((s + 16,), f32),
                pltpu.VMEM((s,), jnp.int32),
                pltpu.VMEM((208,), jnp.int32),
                pltpu.VMEM((208,), f32),
                pltpu.SemaphoreType.DMA((4,)),
                pltpu.SemaphoreType.DMA((2,)),
            ],
        )(a.reshape(-1), score_dense.reshape(-1))
        return _tail(a, e, v, wot3, s=s, hid=hid, nheads=nheads, d=d,
                     rb=rb, nrb=nrb, f32=f32, interpret=interpret)
    cfl = (cache // rb3) * rb3
    if s >= 2048:
        regions = [(0, cfl, 512, 0, False),
                   (cfl, 512, 512, 512, True),
                   (512, 1024, 1024, 512, True),
                   (1024, 1536, 1536, 1024, True),
                   (1536, s, s, 1536, True)]
    else:
        regions = [(0, cfl, s, 0, False), (cfl, s, s, s, True)]
    score_st = jnp.zeros((nheads, s), f32)
    e = jnp.zeros((nheads, s), jnp.int32)
    for (t0, t1, w, w_prev, do_evict) in regions:
        e, score_st = pl.pallas_call(
            functools.partial(_evict_body, s=s, h=nheads, rb=rb3, w=w,
                              w_prev=w_prev, t0=t0, recent=recent, cache=cache,
                              do_evict=do_evict),
            grid=((t1 - t0) // rb3,),
            in_specs=[
                pl.BlockSpec((nheads, rb3, w), lambda tb, t0b=t0 // rb3: (0, t0b + tb, 0)),
                pl.BlockSpec((nheads, w), lambda tb: (0, 0)),
                pl.BlockSpec((nheads, w), lambda tb: (0, 0)),
            ],
            out_specs=[
                pl.BlockSpec((nheads, w), lambda tb: (0, 0)),
                pl.BlockSpec((nheads, w), lambda tb: (0, 0)),
            ],
            out_shape=[jax.ShapeDtypeStruct((nheads, s), jnp.int32),
                       jax.ShapeDtypeStruct((nheads, s), f32)],
            interpret=interpret,
        )(a, score_st, e)
    return _tail(a, e, v, wot3, s=s, hid=hid, nheads=nheads, d=d,
                 rb=rb, nrb=nrb, f32=f32, interpret=interpret)


def _tail(a, e, v, wot3, *, s, hid, nheads, d, rb, nrb, f32, interpret):
    o = pl.pallas_call(
        functools.partial(_attnv_body, s=s, h=nheads, rb=rb),
        grid=(nheads, nrb),
        in_specs=[
            pl.BlockSpec((1, rb, s), lambda hh, i: (hh, i, 0)),
            pl.BlockSpec((nheads, s), lambda hh, i: (0, 0)),
            pl.BlockSpec((1, s, d), lambda hh, i: (hh, 0, 0)),
        ],
        out_specs=pl.BlockSpec((1, rb, d), lambda hh, i: (hh, i, 0)),
        out_shape=jax.ShapeDtypeStruct((nheads, s, d), f32),
        interpret=interpret,
    )(a, e, v)

    y = pl.pallas_call(
        _outproj_body,
        grid=(nrb, nheads),
        in_specs=[
            pl.BlockSpec((1, rb, d), lambda i, hh: (hh, i, 0)),
            pl.BlockSpec((1, d, hid), lambda i, hh: (hh, 0, 0)),
        ],
        out_specs=pl.BlockSpec((rb, hid), lambda i, hh: (i, 0)),
        out_shape=jax.ShapeDtypeStruct((s, hid), f32),
        interpret=interpret,
    )(o, wot3)
    return y


def kernel(hidden_states, attention_mask, position_ids, Wq, Wk, Wv, Wo):
    b, s, hid = hidden_states.shape
    d = 64
    nheads = hid // d
    y = _run(hidden_states[0], Wq, Wk, Wv, Wo, s=s, hid=hid, nheads=nheads, d=d)
    return y.reshape(b, s, hid)


# speedup vs baseline: 1.0531x; 1.0531x over previous
import functools

import jax
import jax.numpy as jnp
import numpy as np
from jax import lax
from jax.experimental import pallas as pl
from jax.experimental.pallas import tpu as pltpu
from jax.experimental.pallas import tpu_sc as plsc

PENALTY = 0.99
NEG = float(np.finfo(np.float32).min)


def _rot_half(x, d):
    h = d // 2
    return jnp.concatenate([-x[:, h:], x[:, :h]], axis=1)


def _proj_body(h_ref, wq_ref, wk_ref, wv_ref, q_ref, k_ref, v_ref):
    h = h_ref[...]
    dn = (((1,), (1,)), ((), ()))
    q_ref[0] = jax.lax.dot_general(h, wq_ref[0], dn, preferred_element_type=jnp.float32)
    k_ref[0] = jax.lax.dot_general(h, wk_ref[0], dn, preferred_element_type=jnp.float32)
    v_ref[0] = jax.lax.dot_general(h, wv_ref[0], dn, preferred_element_type=jnp.float32)


def _scores_body(q_ref, k_ref, cq_ref, sq_ref, ck_ref, sk_ref, a_ref, *, d, scale):
    q = q_ref[0]
    k = k_ref[0]
    qr = q * cq_ref[...] + _rot_half(q, d) * sq_ref[...]
    kr = k * ck_ref[...] + _rot_half(k, d) * sk_ref[...]
    dn = (((1,), (1,)), ((), ()))
    a_ref[0] = jax.lax.dot_general(qr, kr, dn, preferred_element_type=jnp.float32) * scale


def _evict_body(a_ref, sc_in_ref, e_in_ref, e_ref, score_ref, *,
                s, h, rb, w, w_prev, t0, recent, cache, do_evict):
    tb = pl.program_id(0)
    col = jax.lax.broadcasted_iota(jnp.int32, (h, w), 1)

    @pl.when(tb == 0)
    def _init():
        score_ref[...] = jnp.where(col >= w_prev, 0.0, sc_in_ref[...])
        e_ref[...] = jnp.where(col >= w_prev, s + 1, e_in_ref[...])

    for i in range(rb):
        t = t0 + tb * rb + i
        e = e_ref[...]
        score = score_ref[...]
        active = (col <= t) & (e > t)
        ex = jnp.exp(jnp.where(active, a_ref[:, i, :], NEG))
        z = jnp.sum(ex, axis=1, keepdims=True)
        score = jnp.where(active, PENALTY * score + ex / z, 0.0)
        if do_evict:
            cand = (e > t) & (col <= t - recent)
            sc = jnp.where(cand, score, jnp.inf)
            mn = jnp.min(sc, axis=1, keepdims=True)
            evict = jnp.max(jnp.where(cand & (sc == mn), col, -1),
                            axis=1, keepdims=True)
            do = jnp.logical_and(t >= cache, t < s - 1)
            e_ref[...] = jnp.where(jnp.logical_and(do, col == evict), t + 1, e)
        score_ref[...] = score


def _perm(v, idx):
    dn = lax.GatherDimensionNumbers(offset_dims=(), collapsed_slice_dims=(0,),
                                    start_index_map=(0,))
    return lax.gather(v, idx[:, None], dn, (1,),
                      mode=lax.GatherScatterMode.PROMISE_IN_BOUNDS)


def _allred(v, op, lane):
    for k in (8, 4, 2, 1):
        v = op(v, _perm(v, lax.rem(lane + k, 16)))
    return v


def _sc_evict_body(af_hbm, sf_hbm, e_hbm, wrow, score, eloc,
                   gi0, gi1, gb0, gb1, wsem, gsem, *,
                   s, nheads, heavy, recent, cache, nc):
    f32 = jnp.float32
    i32 = jnp.int32
    NBUF = 4
    PREF = 2
    NH = (heavy + 15) // 16
    NR = (recent + 16) // 16
    wid = lax.axis_index("s") * nc + lax.axis_index("c")
    h = wid

    @pl.when(h < nheads)
    def _body():
        lane = lax.broadcasted_iota(i32, (16,), 0)
        zi = jnp.zeros((16,), i32)
        zf = jnp.zeros((16,), f32)
        INF = jnp.full((16,), np.inf, f32)
        NEG1 = jnp.full((16,), -1, i32)

        SL = 512
        pltpu.sync_copy(sf_hbm.at[pl.ds(h * s, SL)], score.at[pl.ds(0, SL)])

        def init_s(j, c):
            score[pl.ds(SL + 16 * j, 16)] = zf
            return c
        lax.fori_loop(0, (s + 16 - SL) // 16, init_s, 0)

        def init_e(j, c):
            eloc[pl.ds(16 * j, 16)] = zi + (s + 1)
            return c
        lax.fori_loop(0, s // 16, init_e, 0)

        def row_off(t):
            return (h * s + t) * s

        WL = 224

        def wst_of(t):
            return jnp.minimum(((t - recent - 8) // 8) * 8, s - WL)

        def wdma_in(t):
            b = lax.rem(t, NBUF)
            pltpu.make_async_copy(af_hbm.at[pl.ds(row_off(t) + wst_of(t), WL)],
                                  wrow.at[pl.ds(b * 256, WL)], wsem.at[b]).start()

        def wwait_in(t):
            b = lax.rem(t, NBUF)
            pltpu.make_async_copy(af_hbm.at[pl.ds(row_off(t) + wst_of(t), WL)],
                                  wrow.at[pl.ds(b * 256, WL)], wsem.at[b]).wait()

        def gissue(t1, hidx):
            gbase = (h * s + t1) * s
            for j in range(NH):
                g = gbase + hidx[j]
                if j < 7:
                    gi0[pl.ds(16 * j, 16)] = g
                else:
                    gi1[pl.ds(16 * (j - 7), 16)] = g
            pltpu.make_async_copy(af_hbm.at[gi0], gb0, gsem.at[0]).start()
            pltpu.make_async_copy(af_hbm.at[gi1], gb1, gsem.at[1]).start()

        def gwait():
            pltpu.make_async_copy(af_hbm.at[gi0], gb0, gsem.at[0]).wait()
            pltpu.make_async_copy(af_hbm.at[gi1], gb1, gsem.at[1]).wait()

        hidx0 = tuple(16 * j + lane for j in range(NH))
        hsc0 = tuple(score[pl.ds(16 * j, 16)] for j in range(NH))
        wdma_in(cache)
        wdma_in(cache + 1)
        gissue(cache, hidx0)

        def step(t, carry):
            hidx, hsc = carry
            wwait_in(t)
            gwait()
            b = lax.rem(t, NBUF)

            @pl.when(t + PREF < s - 1)
            def _():
                wdma_in(t + PREF)

            @pl.when(t + 1 < s - 1)
            def _():
                gissue(t + 1, hidx)

            base = t - recent
            wst = wst_of(t)
            off = base - wst
            exr = []
            zv = zf
            for j in range(NR):
                a = wrow[pl.ds(b * 256 + off + 16 * j, 16)]
                if 16 * (j + 1) <= recent + 1:
                    ex = jnp.exp(a)
                else:
                    ex = jnp.where(16 * j + lane < recent + 1, jnp.exp(a), zf)
                exr.append(ex)
                zv = zv + ex
            pg = base - 1 - wst
            pgal = (pg // 16) * 16
            lv = wrow[pl.ds(b * 256 + pgal, 16)]
            patch = _perm(lv, zi + (pg - pgal))
            gprev = zi + (base - 1)
            exh = []
            for j in range(NH):
                if j < 7:
                    ah = gb0[pl.ds(16 * j, 16)]
                else:
                    ah = gb1[pl.ds(16 * (j - 7), 16)]
                ah = jnp.where(hidx[j] == gprev, patch, ah)
                if 16 * (j + 1) <= heavy:
                    ex = jnp.exp(ah)
                else:
                    ex = jnp.where(16 * j + lane < heavy, jnp.exp(ah), zf)
                exh.append(ex)
                zv = zv + ex
            rz = (zf + 1.0) / _allred(zv, jnp.add, lane)
            gvec = None
            for j in range(NR):
                sl = pl.ds(base + 16 * j, 16)
                old = score[sl]
                if 16 * (j + 1) <= recent + 1:
                    new = PENALTY * old + exr[j] * rz
                else:
                    new = jnp.where(16 * j + lane < recent + 1,
                                    PENALTY * old + exr[j] * rz, old)
                score[sl] = new
                if j == 0:
                    gvec = new
            hsc2 = tuple(PENALTY * hsc[j] + exh[j] * rz for j in range(NH))
            gcand = jnp.where(lane == 0, gvec, INF)
            mv = gcand
            for j in range(NH):
                if 16 * (j + 1) <= heavy:
                    mv = jnp.minimum(mv, hsc2[j])
                else:
                    mv = jnp.minimum(mv, jnp.where(16 * j + lane < heavy,
                                                   hsc2[j], INF))
            mval = _allred(mv, jnp.minimum, lane)
            pv = jnp.where((lane == 0) & (gcand == mval), zi + base, NEG1)
            for j in range(NH):
                if 16 * (j + 1) <= heavy:
                    hit = hsc2[j] == mval
                else:
                    hit = (16 * j + lane < heavy) & (hsc2[j] == mval)
                pv = jnp.maximum(pv, jnp.where(hit, hidx[j], NEG1))
            ev = _allred(pv, jnp.maximum, lane)
            sg = _perm(gvec, lane * 0)
            hidx2 = tuple(jnp.where(hidx[j] == ev, zi + base, hidx[j])
                          for j in range(NH))
            hsc3 = tuple(jnp.where(hidx[j] == ev, sg, hsc2[j])
                         for j in range(NH))
            evs = ev[0]
            al = (evs // 16) * 16
            sl = pl.ds(al, 16)
            eloc[sl] = jnp.where(al + lane == evs, zi + (t + 1), eloc[sl])
            return (hidx2, hsc3)

        lax.fori_loop(cache, s - 1, step, (hidx0, hsc0))
        pltpu.sync_copy(eloc, e_hbm.at[h])


def _attnv_body(a_ref, e_ref, v_ref, o_ref, *, s, h, rb):
    hh = pl.program_id(0)
    rbi = pl.program_id(1)
    a = a_ref[0]
    e_full = e_ref[...]
    hrow = jax.lax.broadcasted_iota(jnp.int32, (h, s), 0)
    e_h = jnp.max(jnp.where(hrow == hh, e_full, 0), axis=0, keepdims=True)
    row = rbi * rb + jax.lax.broadcasted_iota(jnp.int32, (rb, s), 0)
    col = jax.lax.broadcasted_iota(jnp.int32, (rb, s), 1)
    msk = (col <= row) & (row < e_h)
    aa = jnp.where(msk, a, NEG)
    m = jnp.max(aa, axis=1, keepdims=True)
    p = jnp.exp(aa - m)
    p = p / jnp.sum(p, axis=1, keepdims=True)
    dn = (((1,), (0,)), ((), ()))
    o_ref[0] = jax.lax.dot_general(p, v_ref[0], dn, preferred_element_type=jnp.float32)


def _outproj_body(o_ref, wot_ref, y_ref):
    hh = pl.program_id(1)

    @pl.when(hh == 0)
    def _init():
        y_ref[...] = jnp.zeros_like(y_ref)

    dn = (((1,), (0,)), ((), ()))
    y_ref[...] += jax.lax.dot_general(o_ref[0], wot_ref[0], dn, preferred_element_type=jnp.float32)


def _run(hs, Wq, Wk, Wv, Wo, *, s, hid, nheads, d, interpret=False):
    heavy = int(0.1 * s)
    recent = int(0.1 * s)
    cache = heavy + recent
    scale = 1.0 / float(np.sqrt(d).astype(np.float32))
    rb = min(256, s)
    nrb = s // rb
    rb3 = 8
    f32 = jnp.float32

    inv_freq = 1.0 / (10000.0 ** (jnp.arange(0, d, 2, dtype=f32) / d))
    t_ar = jnp.arange(s, dtype=f32)
    freqs = jnp.einsum('i,j->ij', t_ar, inv_freq)
    emb = jnp.concatenate([freqs, freqs], axis=-1)
    cos, sin = jnp.cos(emb), jnp.sin(emb)

    wq3 = Wq.reshape(nheads, d, hid)
    wk3 = Wk.reshape(nheads, d, hid)
    wv3 = Wv.reshape(nheads, d, hid)
    wot3 = Wo.T.reshape(nheads, d, hid)

    q, k, v = pl.pallas_call(
        _proj_body,
        grid=(nheads, nrb),
        in_specs=[
            pl.BlockSpec((rb, hid), lambda hh, i: (i, 0)),
            pl.BlockSpec((1, d, hid), lambda hh, i: (hh, 0, 0)),
            pl.BlockSpec((1, d, hid), lambda hh, i: (hh, 0, 0)),
            pl.BlockSpec((1, d, hid), lambda hh, i: (hh, 0, 0)),
        ],
        out_specs=[
            pl.BlockSpec((1, rb, d), lambda hh, i: (hh, i, 0)),
            pl.BlockSpec((1, rb, d), lambda hh, i: (hh, i, 0)),
            pl.BlockSpec((1, rb, d), lambda hh, i: (hh, i, 0)),
        ],
        out_shape=[jax.ShapeDtypeStruct((nheads, s, d), f32)] * 3,
        interpret=interpret,
    )(hs, wq3, wk3, wv3)

    a = pl.pallas_call(
        functools.partial(_scores_body, d=d, scale=scale),
        grid=(nheads, nrb),
        in_specs=[
            pl.BlockSpec((1, rb, d), lambda hh, i: (hh, i, 0)),
            pl.BlockSpec((1, s, d), lambda hh, i: (hh, 0, 0)),
            pl.BlockSpec((rb, d), lambda hh, i: (i, 0)),
            pl.BlockSpec((rb, d), lambda hh, i: (i, 0)),
            pl.BlockSpec((s, d), lambda hh, i: (0, 0)),
            pl.BlockSpec((s, d), lambda hh, i: (0, 0)),
        ],
        out_specs=pl.BlockSpec((1, rb, s), lambda hh, i: (hh, i, 0)),
        out_shape=jax.ShapeDtypeStruct((nheads, s, s), f32),
        interpret=interpret,
    )(q, k, cos, sin, cos, sin)

    if not interpret and s >= 2048:
        score0 = jnp.zeros((nheads, s), f32)
        e0 = jnp.zeros((nheads, s), jnp.int32)
        cfl = (cache // rb3) * rb3
        _, score_dense = pl.pallas_call(
            functools.partial(_evict_body, s=s, h=nheads, rb=rb3, w=512,
                              w_prev=0, t0=0, recent=recent, cache=cache,
                              do_evict=False),
            grid=(cfl // rb3,),
            in_specs=[
                pl.BlockSpec((nheads, rb3, 512), lambda tb: (0, tb, 0)),
                pl.BlockSpec((nheads, 512), lambda tb: (0, 0)),
                pl.BlockSpec((nheads, 512), lambda tb: (0, 0)),
            ],
            out_specs=[
                pl.BlockSpec((nheads, 512), lambda tb: (0, 0)),
                pl.BlockSpec((nheads, 512), lambda tb: (0, 0)),
            ],
            out_shape=[jax.ShapeDtypeStruct((nheads, s), jnp.int32),
                       jax.ShapeDtypeStruct((nheads, s), f32)],
            interpret=interpret,
        )(a, score0, e0)
        info = plsc.get_sparse_core_info()
        mesh = plsc.VectorSubcoreMesh(core_axis_name="c", subcore_axis_name="s")
        e = pl.kernel(
            functools.partial(_sc_evict_body, s=s, nheads=nheads, heavy=heavy,
                              recent=recent, cache=cache, nc=info.num_cores),
            mesh=mesh,
            out_type=jax.ShapeDtypeStruct((nheads, s), jnp.int32),
            scratch_types=[
                pltpu.VMEM((4 * 256,), f32),
                pltpu.VMEM((s + 16,), f32),
                pltpu.VMEM((s,), jnp.int32),
                pltpu.VMEM((112,), jnp.int32),
                pltpu.VMEM((96,), jnp.int32),
                pltpu.VMEM((112,), f32),
                pltpu.VMEM((96,), f32),
                pltpu.SemaphoreType.DMA((4,)),
                pltpu.SemaphoreType.DMA((2,)),
            ],
        )(a.reshape(-1), score_dense.reshape(-1))
        return _tail(a, e, v, wot3, s=s, hid=hid, nheads=nheads, d=d,
                     rb=rb, nrb=nrb, f32=f32, interpret=interpret)
    cfl = (cache // rb3) * rb3
    if s >= 2048:
        regions = [(0, cfl, 512, 0, False),
                   (cfl, 512, 512, 512, True),
                   (512, 1024, 1024, 512, True),
                   (1024, 1536, 1536, 1024, True),
                   (1536, s, s, 1536, True)]
    else:
        regions = [(0, cfl, s, 0, False), (cfl, s, s, s, True)]
    score_st = jnp.zeros((nheads, s), f32)
    e = jnp.zeros((nheads, s), jnp.int32)
    for (t0, t1, w, w_prev, do_evict) in regions:
        e, score_st = pl.pallas_call(
            functools.partial(_evict_body, s=s, h=nheads, rb=rb3, w=w,
                              w_prev=w_prev, t0=t0, recent=recent, cache=cache,
                              do_evict=do_evict),
            grid=((t1 - t0) // rb3,),
            in_specs=[
                pl.BlockSpec((nheads, rb3, w), lambda tb, t0b=t0 // rb3: (0, t0b + tb, 0)),
                pl.BlockSpec((nheads, w), lambda tb: (0, 0)),
                pl.BlockSpec((nheads, w), lambda tb: (0, 0)),
            ],
            out_specs=[
                pl.BlockSpec((nheads, w), lambda tb: (0, 0)),
                pl.BlockSpec((nheads, w), lambda tb: (0, 0)),
            ],
            out_shape=[jax.ShapeDtypeStruct((nheads, s), jnp.int32),
                       jax.ShapeDtypeStruct((nheads, s), f32)],
            interpret=interpret,
        )(a, score_st, e)
    return _tail(a, e, v, wot3, s=s, hid=hid, nheads=nheads, d=d,
                 rb=rb, nrb=nrb, f32=f32, interpret=interpret)


def _tail(a, e, v, wot3, *, s, hid, nheads, d, rb, nrb, f32, interpret):
    o = pl.pallas_call(
        functools.partial(_attnv_body, s=s, h=nheads, rb=rb),
        grid=(nheads, nrb),
        in_specs=[
            pl.BlockSpec((1, rb, s), lambda hh, i: (hh, i, 0)),
            pl.BlockSpec((nheads, s), lambda hh, i: (0, 0)),
            pl.BlockSpec((1, s, d), lambda hh, i: (hh, 0, 0)),
        ],
        out_specs=pl.BlockSpec((1, rb, d), lambda hh, i: (hh, i, 0)),
        out_shape=jax.ShapeDtypeStruct((nheads, s, d), f32),
        interpret=interpret,
    )(a, e, v)

    y = pl.pallas_call(
        _outproj_body,
        grid=(nrb, nheads),
        in_specs=[
            pl.BlockSpec((1, rb, d), lambda i, hh: (hh, i, 0)),
            pl.BlockSpec((1, d, hid), lambda i, hh: (hh, 0, 0)),
        ],
        out_specs=pl.BlockSpec((rb, hid), lambda i, hh: (i, 0)),
        out_shape=jax.ShapeDtypeStruct((s, hid), f32),
        interpret=interpret,
    )(o, wot3)
    return y


def kernel(hidden_states, attention_mask, position_ids, Wq, Wk, Wv, Wo):
    b, s, hid = hidden_states.shape
    d = 64
    nheads = hid // d
    y = _run(hidden_states[0], Wq, Wk, Wv, Wo, s=s, hid=hid, nheads=nheads, d=d)
    return y.reshape(b, s, hid)


# speedup vs baseline: 1.6383x; 1.5558x over previous
import functools

import jax
import jax.numpy as jnp
import numpy as np
from jax import lax
from jax.experimental import pallas as pl
from jax.experimental.pallas import tpu as pltpu
from jax.experimental.pallas import tpu_sc as plsc

PENALTY = 0.99
NEG = float(np.finfo(np.float32).min)


def _rot_half(x, d):
    h = d // 2
    return jnp.concatenate([-x[:, h:], x[:, :h]], axis=1)


def _proj_body(h_ref, wq_ref, wk_ref, wv_ref, q_ref, k_ref, v_ref):
    h = h_ref[...]
    dn = (((1,), (1,)), ((), ()))
    q_ref[0] = jax.lax.dot_general(h, wq_ref[0], dn, preferred_element_type=jnp.float32)
    k_ref[0] = jax.lax.dot_general(h, wk_ref[0], dn, preferred_element_type=jnp.float32)
    v_ref[0] = jax.lax.dot_general(h, wv_ref[0], dn, preferred_element_type=jnp.float32)


def _scores_body(q_ref, k_ref, cq_ref, sq_ref, ck_ref, sk_ref, a_ref, *, d, scale):
    q = q_ref[0]
    k = k_ref[0]
    qr = q * cq_ref[...] + _rot_half(q, d) * sq_ref[...]
    kr = k * ck_ref[...] + _rot_half(k, d) * sk_ref[...]
    dn = (((1,), (1,)), ((), ()))
    a_ref[0] = jax.lax.dot_general(qr, kr, dn, preferred_element_type=jnp.float32) * scale


def _evict_body(a_ref, sc_in_ref, e_in_ref, e_ref, score_ref, *,
                s, h, rb, w, w_prev, t0, recent, cache, do_evict):
    tb = pl.program_id(0)
    col = jax.lax.broadcasted_iota(jnp.int32, (h, w), 1)

    @pl.when(tb == 0)
    def _init():
        score_ref[...] = jnp.where(col >= w_prev, 0.0, sc_in_ref[...])
        e_ref[...] = jnp.where(col >= w_prev, s + 1, e_in_ref[...])

    for i in range(rb):
        t = t0 + tb * rb + i
        e = e_ref[...]
        score = score_ref[...]
        active = (col <= t) & (e > t)
        ex = jnp.exp(jnp.where(active, a_ref[:, i, :], NEG))
        z = jnp.sum(ex, axis=1, keepdims=True)
        score = jnp.where(active, PENALTY * score + ex / z, 0.0)
        if do_evict:
            cand = (e > t) & (col <= t - recent)
            sc = jnp.where(cand, score, jnp.inf)
            mn = jnp.min(sc, axis=1, keepdims=True)
            evict = jnp.max(jnp.where(cand & (sc == mn), col, -1),
                            axis=1, keepdims=True)
            do = jnp.logical_and(t >= cache, t < s - 1)
            e_ref[...] = jnp.where(jnp.logical_and(do, col == evict), t + 1, e)
        score_ref[...] = score


def _perm(v, idx):
    dn = lax.GatherDimensionNumbers(offset_dims=(), collapsed_slice_dims=(0,),
                                    start_index_map=(0,))
    return lax.gather(v, idx[:, None], dn, (1,),
                      mode=lax.GatherScatterMode.PROMISE_IN_BOUNDS)


def _allred(v, op, lane):
    for k in (8, 4, 2, 1):
        v = op(v, _perm(v, lax.rem(lane + k, 16)))
    return v


def _sc_evict_body(af_hbm, sf_hbm, e_hbm, wrow, score, eloc,
                   gi0, gi1, gb0, gb1, wsem, gsem, *,
                   s, nheads, heavy, recent, cache, nc):
    f32 = jnp.float32
    i32 = jnp.int32
    NBUF = 4
    PREF = 2
    NH = (heavy + 15) // 16
    NR = (recent + 16) // 16
    wid = lax.axis_index("s") * nc + lax.axis_index("c")
    h = wid

    @pl.when(h < nheads)
    def _body():
        lane = lax.broadcasted_iota(i32, (16,), 0)
        zi = jnp.zeros((16,), i32)
        zf = jnp.zeros((16,), f32)
        INF = jnp.full((16,), np.inf, f32)
        NEG1 = jnp.full((16,), -1, i32)

        SL = 512
        pltpu.sync_copy(sf_hbm.at[pl.ds(h * s, SL)], score.at[pl.ds(0, SL)])

        def init_s(j, c):
            score[pl.ds(SL + 16 * j, 16)] = zf
            return c
        lax.fori_loop(0, (s + 16 - SL) // 16, init_s, 0)

        def init_e(j, c):
            eloc[pl.ds(16 * j, 16)] = zi + (s + 1)
            return c
        lax.fori_loop(0, s // 16, init_e, 0)

        def row_off(t):
            return (h * s + t) * s

        WL = 224

        def wst_of(t):
            return jnp.minimum(((t - recent - 8) // 8) * 8, s - WL)

        def wdma_in(t):
            b = lax.rem(t, NBUF)
            pltpu.make_async_copy(af_hbm.at[pl.ds(row_off(t) + wst_of(t), WL)],
                                  wrow.at[pl.ds(b * 256, WL)], wsem.at[b]).start()

        def wwait_in(t):
            b = lax.rem(t, NBUF)
            pltpu.make_async_copy(af_hbm.at[pl.ds(row_off(t) + wst_of(t), WL)],
                                  wrow.at[pl.ds(b * 256, WL)], wsem.at[b]).wait()

        def gissue(t1, hidx):
            p = lax.rem(t1, 2)
            gbase = (h * s + t1) * s
            for j in range(NH):
                g = gbase + hidx[j]
                if j < 7:
                    gi0[pl.ds(112 * p + 16 * j, 16)] = g
                else:
                    gi1[pl.ds(96 * p + 16 * (j - 7), 16)] = g
            pltpu.make_async_copy(af_hbm.at[gi0.at[pl.ds(112 * p, 112)]],
                                  gb0.at[pl.ds(112 * p, 112)],
                                  gsem.at[2 * p]).start()
            pltpu.make_async_copy(af_hbm.at[gi1.at[pl.ds(96 * p, 96)]],
                                  gb1.at[pl.ds(96 * p, 96)],
                                  gsem.at[2 * p + 1]).start()

        def gwait(t1):
            p = lax.rem(t1, 2)
            pltpu.make_async_copy(af_hbm.at[gi0.at[pl.ds(112 * p, 112)]],
                                  gb0.at[pl.ds(112 * p, 112)],
                                  gsem.at[2 * p]).wait()
            pltpu.make_async_copy(af_hbm.at[gi1.at[pl.ds(96 * p, 96)]],
                                  gb1.at[pl.ds(96 * p, 96)],
                                  gsem.at[2 * p + 1]).wait()

        hidx0 = tuple(16 * j + lane for j in range(NH))
        hsc0 = tuple(score[pl.ds(16 * j, 16)] for j in range(NH))
        wdma_in(cache)
        wdma_in(cache + 1)
        gissue(cache, hidx0)
        gissue(cache + 1, hidx0)

        def step(t, carry):
            hidx, hsc = carry
            wwait_in(t)
            gwait(t)
            b = lax.rem(t, NBUF)
            p2 = lax.rem(t, 2)

            @pl.when(t + PREF < s - 1)
            def _():
                wdma_in(t + PREF)

            @pl.when(t + 2 < s - 1)
            def _():
                gissue(t + 2, hidx)

            base = t - recent
            wst = wst_of(t)
            off = base - wst
            exr = []
            zv = zf
            for j in range(NR):
                a = wrow[pl.ds(b * 256 + off + 16 * j, 16)]
                if 16 * (j + 1) <= recent + 1:
                    ex = jnp.exp(a)
                else:
                    ex = jnp.where(16 * j + lane < recent + 1, jnp.exp(a), zf)
                exr.append(ex)
                zv = zv + ex
            pg1 = base - 1 - wst
            pga1 = (pg1 // 16) * 16
            patch1 = _perm(wrow[pl.ds(b * 256 + pga1, 16)], zi + (pg1 - pga1))
            pg2 = base - 2 - wst
            pga2 = (pg2 // 16) * 16
            patch2 = _perm(wrow[pl.ds(b * 256 + pga2, 16)], zi + (pg2 - pga2))
            gprev1 = zi + (base - 1)
            gprev2 = zi + (base - 2)
            exh = []
            for j in range(NH):
                if j < 7:
                    ah = gb0[pl.ds(112 * p2 + 16 * j, 16)]
                else:
                    ah = gb1[pl.ds(96 * p2 + 16 * (j - 7), 16)]
                ah = jnp.where(hidx[j] == gprev1, patch1,
                               jnp.where(hidx[j] == gprev2, patch2, ah))
                if 16 * (j + 1) <= heavy:
                    ex = jnp.exp(ah)
                else:
                    ex = jnp.where(16 * j + lane < heavy, jnp.exp(ah), zf)
                exh.append(ex)
                zv = zv + ex
            rz = (zf + 1.0) / _allred(zv, jnp.add, lane)
            gvec = None
            for j in range(NR):
                sl = pl.ds(base + 16 * j, 16)
                old = score[sl]
                if 16 * (j + 1) <= recent + 1:
                    new = PENALTY * old + exr[j] * rz
                else:
                    new = jnp.where(16 * j + lane < recent + 1,
                                    PENALTY * old + exr[j] * rz, old)
                score[sl] = new
                if j == 0:
                    gvec = new
            hsc2 = tuple(PENALTY * hsc[j] + exh[j] * rz for j in range(NH))
            gcand = jnp.where(lane == 0, gvec, INF)
            mv = gcand
            for j in range(NH):
                if 16 * (j + 1) <= heavy:
                    mv = jnp.minimum(mv, hsc2[j])
                else:
                    mv = jnp.minimum(mv, jnp.where(16 * j + lane < heavy,
                                                   hsc2[j], INF))
            mval = _allred(mv, jnp.minimum, lane)
            pv = jnp.where((lane == 0) & (gcand == mval), zi + base, NEG1)
            for j in range(NH):
                if 16 * (j + 1) <= heavy:
                    hit = hsc2[j] == mval
                else:
                    hit = (16 * j + lane < heavy) & (hsc2[j] == mval)
                pv = jnp.maximum(pv, jnp.where(hit, hidx[j], NEG1))
            ev = _allred(pv, jnp.maximum, lane)
            sg = _perm(gvec, lane * 0)
            hidx2 = tuple(jnp.where(hidx[j] == ev, zi + base, hidx[j])
                          for j in range(NH))
            hsc3 = tuple(jnp.where(hidx[j] == ev, sg, hsc2[j])
                         for j in range(NH))
            evs = ev[0]
            al = (evs // 16) * 16
            sl = pl.ds(al, 16)
            eloc[sl] = jnp.where(al + lane == evs, zi + (t + 1), eloc[sl])
            return (hidx2, hsc3)

        lax.fori_loop(cache, s - 1, step, (hidx0, hsc0))
        pltpu.sync_copy(eloc, e_hbm.at[h])


def _attnv_body(a_ref, e_ref, v_ref, o_ref, *, s, h, rb):
    hh = pl.program_id(0)
    rbi = pl.program_id(1)
    a = a_ref[0]
    e_full = e_ref[...]
    hrow = jax.lax.broadcasted_iota(jnp.int32, (h, s), 0)
    e_h = jnp.max(jnp.where(hrow == hh, e_full, 0), axis=0, keepdims=True)
    row = rbi * rb + jax.lax.broadcasted_iota(jnp.int32, (rb, s), 0)
    col = jax.lax.broadcasted_iota(jnp.int32, (rb, s), 1)
    msk = (col <= row) & (row < e_h)
    aa = jnp.where(msk, a, NEG)
    m = jnp.max(aa, axis=1, keepdims=True)
    p = jnp.exp(aa - m)
    p = p / jnp.sum(p, axis=1, keepdims=True)
    dn = (((1,), (0,)), ((), ()))
    o_ref[0] = jax.lax.dot_general(p, v_ref[0], dn, preferred_element_type=jnp.float32)


def _outproj_body(o_ref, wot_ref, y_ref):
    hh = pl.program_id(1)

    @pl.when(hh == 0)
    def _init():
        y_ref[...] = jnp.zeros_like(y_ref)

    dn = (((1,), (0,)), ((), ()))
    y_ref[...] += jax.lax.dot_general(o_ref[0], wot_ref[0], dn, preferred_element_type=jnp.float32)


def _run(hs, Wq, Wk, Wv, Wo, *, s, hid, nheads, d, interpret=False):
    heavy = int(0.1 * s)
    recent = int(0.1 * s)
    cache = heavy + recent
    scale = 1.0 / float(np.sqrt(d).astype(np.float32))
    rb = min(256, s)
    nrb = s // rb
    rb3 = 8
    f32 = jnp.float32

    inv_freq = 1.0 / (10000.0 ** (jnp.arange(0, d, 2, dtype=f32) / d))
    t_ar = jnp.arange(s, dtype=f32)
    freqs = jnp.einsum('i,j->ij', t_ar, inv_freq)
    emb = jnp.concatenate([freqs, freqs], axis=-1)
    cos, sin = jnp.cos(emb), jnp.sin(emb)

    wq3 = Wq.reshape(nheads, d, hid)
    wk3 = Wk.reshape(nheads, d, hid)
    wv3 = Wv.reshape(nheads, d, hid)
    wot3 = Wo.T.reshape(nheads, d, hid)

    q, k, v = pl.pallas_call(
        _proj_body,
        grid=(nheads, nrb),
        in_specs=[
            pl.BlockSpec((rb, hid), lambda hh, i: (i, 0)),
            pl.BlockSpec((1, d, hid), lambda hh, i: (hh, 0, 0)),
            pl.BlockSpec((1, d, hid), lambda hh, i: (hh, 0, 0)),
            pl.BlockSpec((1, d, hid), lambda hh, i: (hh, 0, 0)),
        ],
        out_specs=[
            pl.BlockSpec((1, rb, d), lambda hh, i: (hh, i, 0)),
            pl.BlockSpec((1, rb, d), lambda hh, i: (hh, i, 0)),
            pl.BlockSpec((1, rb, d), lambda hh, i: (hh, i, 0)),
        ],
        out_shape=[jax.ShapeDtypeStruct((nheads, s, d), f32)] * 3,
        interpret=interpret,
    )(hs, wq3, wk3, wv3)

    a = pl.pallas_call(
        functools.partial(_scores_body, d=d, scale=scale),
        grid=(nheads, nrb),
        in_specs=[
            pl.BlockSpec((1, rb, d), lambda hh, i: (hh, i, 0)),
            pl.BlockSpec((1, s, d), lambda hh, i: (hh, 0, 0)),
            pl.BlockSpec((rb, d), lambda hh, i: (i, 0)),
            pl.BlockSpec((rb, d), lambda hh, i: (i, 0)),
            pl.BlockSpec((s, d), lambda hh, i: (0, 0)),
            pl.BlockSpec((s, d), lambda hh, i: (0, 0)),
        ],
        out_specs=pl.BlockSpec((1, rb, s), lambda hh, i: (hh, i, 0)),
        out_shape=jax.ShapeDtypeStruct((nheads, s, s), f32),
        interpret=interpret,
    )(q, k, cos, sin, cos, sin)

    if not interpret and s >= 2048:
        score0 = jnp.zeros((nheads, s), f32)
        e0 = jnp.zeros((nheads, s), jnp.int32)
        cfl = (cache // rb3) * rb3
        _, score_dense = pl.pallas_call(
            functools.partial(_evict_body, s=s, h=nheads, rb=rb3, w=512,
                              w_prev=0, t0=0, recent=recent, cache=cache,
                              do_evict=False),
            grid=(cfl // rb3,),
            in_specs=[
                pl.BlockSpec((nheads, rb3, 512), lambda tb: (0, tb, 0)),
                pl.BlockSpec((nheads, 512), lambda tb: (0, 0)),
                pl.BlockSpec((nheads, 512), lambda tb: (0, 0)),
            ],
            out_specs=[
                pl.BlockSpec((nheads, 512), lambda tb: (0, 0)),
                pl.BlockSpec((nheads, 512), lambda tb: (0, 0)),
            ],
            out_shape=[jax.ShapeDtypeStruct((nheads, s), jnp.int32),
                       jax.ShapeDtypeStruct((nheads, s), f32)],
            interpret=interpret,
        )(a, score0, e0)
        info = plsc.get_sparse_core_info()
        mesh = plsc.VectorSubcoreMesh(core_axis_name="c", subcore_axis_name="s")
        e = pl.kernel(
            functools.partial(_sc_evict_body, s=s, nheads=nheads, heavy=heavy,
                              recent=recent, cache=cache, nc=info.num_cores),
            mesh=mesh,
            out_type=jax.ShapeDtypeStruct((nheads, s), jnp.int32),
            scratch_types=[
                pltpu.VMEM((4 * 256,), f32),
                pltpu.VMEM((s + 16,), f32),
                pltpu.VMEM((s,), jnp.int32),
                pltpu.VMEM((224,), jnp.int32),
                pltpu.VMEM((192,), jnp.int32),
                pltpu.VMEM((224,), f32),
                pltpu.VMEM((192,), f32),
                pltpu.SemaphoreType.DMA((4,)),
                pltpu.SemaphoreType.DMA((4,)),
            ],
        )(a.reshape(-1), score_dense.reshape(-1))
        return _tail(a, e, v, wot3, s=s, hid=hid, nheads=nheads, d=d,
                     rb=rb, nrb=nrb, f32=f32, interpret=interpret)
    cfl = (cache // rb3) * rb3
    if s >= 2048:
        regions = [(0, cfl, 512, 0, False),
                   (cfl, 512, 512, 512, True),
                   (512, 1024, 1024, 512, True),
                   (1024, 1536, 1536, 1024, True),
                   (1536, s, s, 1536, True)]
    else:
        regions = [(0, cfl, s, 0, False), (cfl, s, s, s, True)]
    score_st = jnp.zeros((nheads, s), f32)
    e = jnp.zeros((nheads, s), jnp.int32)
    for (t0, t1, w, w_prev, do_evict) in regions:
        e, score_st = pl.pallas_call(
            functools.partial(_evict_body, s=s, h=nheads, rb=rb3, w=w,
                              w_prev=w_prev, t0=t0, recent=recent, cache=cache,
                              do_evict=do_evict),
            grid=((t1 - t0) // rb3,),
            in_specs=[
                pl.BlockSpec((nheads, rb3, w), lambda tb, t0b=t0 // rb3: (0, t0b + tb, 0)),
                pl.BlockSpec((nheads, w), lambda tb: (0, 0)),
                pl.BlockSpec((nheads, w), lambda tb: (0, 0)),
            ],
            out_specs=[
                pl.BlockSpec((nheads, w), lambda tb: (0, 0)),
                pl.BlockSpec((nheads, w), lambda tb: (0, 0)),
            ],
            out_shape=[jax.ShapeDtypeStruct((nheads, s), jnp.int32),
                       jax.ShapeDtypeStruct((nheads, s), f32)],
            interpret=interpret,
        )(a, score_st, e)
    return _tail(a, e, v, wot3, s=s, hid=hid, nheads=nheads, d=d,
                 rb=rb, nrb=nrb, f32=f32, interpret=interpret)


def _tail(a, e, v, wot3, *, s, hid, nheads, d, rb, nrb, f32, interpret):
    o = pl.pallas_call(
        functools.partial(_attnv_body, s=s, h=nheads, rb=rb),
        grid=(nheads, nrb),
        in_specs=[
            pl.BlockSpec((1, rb, s), lambda hh, i: (hh, i, 0)),
            pl.BlockSpec((nheads, s), lambda hh, i: (0, 0)),
            pl.BlockSpec((1, s, d), lambda hh, i: (hh, 0, 0)),
        ],
        out_specs=pl.BlockSpec((1, rb, d), lambda hh, i: (hh, i, 0)),
        out_shape=jax.ShapeDtypeStruct((nheads, s, d), f32),
        interpret=interpret,
    )(a, e, v)

    y = pl.pallas_call(
        _outproj_body,
        grid=(nrb, nheads),
        in_specs=[
            pl.BlockSpec((1, rb, d), lambda i, hh: (hh, i, 0)),
            pl.BlockSpec((1, d, hid), lambda i, hh: (hh, 0, 0)),
        ],
        out_specs=pl.BlockSpec((rb, hid), lambda i, hh: (i, 0)),
        out_shape=jax.ShapeDtypeStruct((s, hid), f32),
        interpret=interpret,
    )(o, wot3)
    return y


def kernel(hidden_states, attention_mask, position_ids, Wq, Wk, Wv, Wo):
    b, s, hid = hidden_states.shape
    d = 64
    nheads = hid // d
    y = _run(hidden_states[0], Wq, Wk, Wv, Wo, s=s, hid=hid, nheads=nheads, d=d)
    return y.reshape(b, s, hid)


# speedup vs baseline: 2.0886x; 1.2748x over previous
import functools

import jax
import jax.numpy as jnp
import numpy as np
from jax import lax
from jax.experimental import pallas as pl
from jax.experimental.pallas import tpu as pltpu
from jax.experimental.pallas import tpu_sc as plsc

PENALTY = 0.99
NEG = float(np.finfo(np.float32).min)


def _rot_half(x, d):
    h = d // 2
    return jnp.concatenate([-x[:, h:], x[:, :h]], axis=1)


def _proj_body(h_ref, wq_ref, wk_ref, wv_ref, q_ref, k_ref, v_ref):
    h = h_ref[...]
    dn = (((1,), (1,)), ((), ()))
    q_ref[0] = jax.lax.dot_general(h, wq_ref[0], dn, preferred_element_type=jnp.float32)
    k_ref[0] = jax.lax.dot_general(h, wk_ref[0], dn, preferred_element_type=jnp.float32)
    v_ref[0] = jax.lax.dot_general(h, wv_ref[0], dn, preferred_element_type=jnp.float32)


def _scores_body(q_ref, k_ref, cq_ref, sq_ref, ck_ref, sk_ref, a_ref, *, d, scale):
    q = q_ref[0]
    k = k_ref[0]
    qr = q * cq_ref[...] + _rot_half(q, d) * sq_ref[...]
    kr = k * ck_ref[...] + _rot_half(k, d) * sk_ref[...]
    dn = (((1,), (1,)), ((), ()))
    a_ref[0] = jax.lax.dot_general(qr, kr, dn, preferred_element_type=jnp.float32) * scale


def _evict_body(a_ref, sc_in_ref, e_in_ref, e_ref, score_ref, *,
                s, h, rb, w, w_prev, t0, recent, cache, do_evict):
    tb = pl.program_id(0)
    col = jax.lax.broadcasted_iota(jnp.int32, (h, w), 1)

    @pl.when(tb == 0)
    def _init():
        score_ref[...] = jnp.where(col >= w_prev, 0.0, sc_in_ref[...])
        e_ref[...] = jnp.where(col >= w_prev, s + 1, e_in_ref[...])

    for i in range(rb):
        t = t0 + tb * rb + i
        e = e_ref[...]
        score = score_ref[...]
        active = (col <= t) & (e > t)
        ex = jnp.exp(jnp.where(active, a_ref[:, i, :], NEG))
        z = jnp.sum(ex, axis=1, keepdims=True)
        score = jnp.where(active, PENALTY * score + ex / z, 0.0)
        if do_evict:
            cand = (e > t) & (col <= t - recent)
            sc = jnp.where(cand, score, jnp.inf)
            mn = jnp.min(sc, axis=1, keepdims=True)
            evict = jnp.max(jnp.where(cand & (sc == mn), col, -1),
                            axis=1, keepdims=True)
            do = jnp.logical_and(t >= cache, t < s - 1)
            e_ref[...] = jnp.where(jnp.logical_and(do, col == evict), t + 1, e)
        score_ref[...] = score


def _perm(v, idx):
    dn = lax.GatherDimensionNumbers(offset_dims=(), collapsed_slice_dims=(0,),
                                    start_index_map=(0,))
    return lax.gather(v, idx[:, None], dn, (1,),
                      mode=lax.GatherScatterMode.PROMISE_IN_BOUNDS)


def _allred(v, op, lane):
    for k in (8, 4, 2, 1):
        v = op(v, _perm(v, lax.rem(lane + k, 16)))
    return v


def _sc_evict_body(af_hbm, sf_hbm, e_hbm, wrow, score, eloc,
                   gi0, gi1, gb0, gb1, wsem, gsem, *,
                   s, nheads, heavy, recent, cache, nc):
    f32 = jnp.float32
    i32 = jnp.int32
    NBUF = 4
    PREF = 2
    NH = (heavy + 15) // 16
    NR = (recent + 16) // 16
    wid = lax.axis_index("s") * nc + lax.axis_index("c")
    h = wid

    @pl.when(h < nheads)
    def _body():
        lane = lax.broadcasted_iota(i32, (16,), 0)
        zi = jnp.zeros((16,), i32)
        zf = jnp.zeros((16,), f32)
        INF = jnp.full((16,), np.inf, f32)
        NEG1 = jnp.full((16,), -1, i32)

        SL = 512
        pltpu.sync_copy(sf_hbm.at[pl.ds(h * s, SL)], score.at[pl.ds(0, SL)])

        def init_s(j, c):
            score[pl.ds(SL + 16 * j, 16)] = zf
            return c
        lax.fori_loop(0, (s + 16 - SL) // 16, init_s, 0)

        def init_e(j, c):
            eloc[pl.ds(16 * j, 16)] = zi + (s + 1)
            return c
        lax.fori_loop(0, s // 16, init_e, 0)

        def row_off(t):
            return (h * s + t) * s

        WL = 224

        def wst_of(t):
            return jnp.minimum(((t - recent - 8) // 8) * 8, s - WL)

        def wdma_in(t):
            b = lax.rem(t, NBUF)
            pltpu.make_async_copy(af_hbm.at[pl.ds(row_off(t) + wst_of(t), WL)],
                                  wrow.at[pl.ds(b * 256, WL)], wsem.at[b]).start()

        def wwait_in(t):
            b = lax.rem(t, NBUF)
            pltpu.make_async_copy(af_hbm.at[pl.ds(row_off(t) + wst_of(t), WL)],
                                  wrow.at[pl.ds(b * 256, WL)], wsem.at[b]).wait()

        def gissue(t1, hidx):
            p = lax.rem(t1, 4)
            gbase = (h * s + t1) * s
            for j in range(NH):
                g = gbase + hidx[j]
                if j < 7:
                    gi0[pl.ds(112 * p + 16 * j, 16)] = g
                else:
                    gi1[pl.ds(96 * p + 16 * (j - 7), 16)] = g
            pltpu.make_async_copy(af_hbm.at[gi0.at[pl.ds(112 * p, 112)]],
                                  gb0.at[pl.ds(112 * p, 112)],
                                  gsem.at[2 * p]).start()
            pltpu.make_async_copy(af_hbm.at[gi1.at[pl.ds(96 * p, 96)]],
                                  gb1.at[pl.ds(96 * p, 96)],
                                  gsem.at[2 * p + 1]).start()

        def gwait(t1):
            p = lax.rem(t1, 4)
            pltpu.make_async_copy(af_hbm.at[gi0.at[pl.ds(112 * p, 112)]],
                                  gb0.at[pl.ds(112 * p, 112)],
                                  gsem.at[2 * p]).wait()
            pltpu.make_async_copy(af_hbm.at[gi1.at[pl.ds(96 * p, 96)]],
                                  gb1.at[pl.ds(96 * p, 96)],
                                  gsem.at[2 * p + 1]).wait()

        hidx0 = tuple(16 * j + lane for j in range(NH))
        hsc0 = tuple(score[pl.ds(16 * j, 16)] for j in range(NH))
        wdma_in(cache)
        wdma_in(cache + 1)
        gissue(cache, hidx0)
        gissue(cache + 1, hidx0)
        gissue(cache + 2, hidx0)
        gissue(cache + 3, hidx0)

        def step(t, carry):
            hidx, hsc = carry
            wwait_in(t)
            gwait(t)
            b = lax.rem(t, NBUF)
            p2 = lax.rem(t, 4)

            @pl.when(t + PREF < s - 1)
            def _():
                wdma_in(t + PREF)

            @pl.when(t + 4 < s - 1)
            def _():
                gissue(t + 4, hidx)

            base = t - recent
            wst = wst_of(t)
            off = base - wst
            exr = []
            zv = zf
            for j in range(NR):
                a = wrow[pl.ds(b * 256 + off + 16 * j, 16)]
                if 16 * (j + 1) <= recent + 1:
                    ex = jnp.exp(a)
                else:
                    ex = jnp.where(16 * j + lane < recent + 1, jnp.exp(a), zf)
                exr.append(ex)
                zv = zv + ex
            pg1 = base - 1 - wst
            pga1 = (pg1 // 16) * 16
            patch1 = _perm(wrow[pl.ds(b * 256 + pga1, 16)], zi + (pg1 - pga1))
            pg2 = base - 2 - wst
            pga2 = (pg2 // 16) * 16
            patch2 = _perm(wrow[pl.ds(b * 256 + pga2, 16)], zi + (pg2 - pga2))
            pg3 = base - 3 - wst
            pga3 = (pg3 // 16) * 16
            patch3 = _perm(wrow[pl.ds(b * 256 + pga3, 16)], zi + (pg3 - pga3))
            pg4 = base - 4 - wst
            pga4 = (pg4 // 16) * 16
            patch4 = _perm(wrow[pl.ds(b * 256 + pga4, 16)], zi + (pg4 - pga4))
            gprev1 = zi + (base - 1)
            gprev2 = zi + (base - 2)
            gprev3 = zi + (base - 3)
            gprev4 = zi + (base - 4)
            exh = []
            for j in range(NH):
                if j < 7:
                    ah = gb0[pl.ds(112 * p2 + 16 * j, 16)]
                else:
                    ah = gb1[pl.ds(96 * p2 + 16 * (j - 7), 16)]
                ah = jnp.where(hidx[j] == gprev1, patch1,
                               jnp.where(hidx[j] == gprev2, patch2,
                               jnp.where(hidx[j] == gprev3, patch3,
                               jnp.where(hidx[j] == gprev4, patch4, ah))))
                if 16 * (j + 1) <= heavy:
                    ex = jnp.exp(ah)
                else:
                    ex = jnp.where(16 * j + lane < heavy, jnp.exp(ah), zf)
                exh.append(ex)
                zv = zv + ex
            rz = (zf + 1.0) / _allred(zv, jnp.add, lane)
            gvec = None
            for j in range(NR):
                sl = pl.ds(base + 16 * j, 16)
                old = score[sl]
                if 16 * (j + 1) <= recent + 1:
                    new = PENALTY * old + exr[j] * rz
                else:
                    new = jnp.where(16 * j + lane < recent + 1,
                                    PENALTY * old + exr[j] * rz, old)
                score[sl] = new
                if j == 0:
                    gvec = new
            hsc2 = tuple(PENALTY * hsc[j] + exh[j] * rz for j in range(NH))
            gcand = jnp.where(lane == 0, gvec, INF)
            mv = gcand
            for j in range(NH):
                if 16 * (j + 1) <= heavy:
                    mv = jnp.minimum(mv, hsc2[j])
                else:
                    mv = jnp.minimum(mv, jnp.where(16 * j + lane < heavy,
                                                   hsc2[j], INF))
            mval = _allred(mv, jnp.minimum, lane)
            pv = jnp.where((lane == 0) & (gcand == mval), zi + base, NEG1)
            for j in range(NH):
                if 16 * (j + 1) <= heavy:
                    hit = hsc2[j] == mval
                else:
                    hit = (16 * j + lane < heavy) & (hsc2[j] == mval)
                pv = jnp.maximum(pv, jnp.where(hit, hidx[j], NEG1))
            ev = _allred(pv, jnp.maximum, lane)
            sg = _perm(gvec, lane * 0)
            hidx2 = tuple(jnp.where(hidx[j] == ev, zi + base, hidx[j])
                          for j in range(NH))
            hsc3 = tuple(jnp.where(hidx[j] == ev, sg, hsc2[j])
                         for j in range(NH))
            evs = ev[0]
            al = (evs // 16) * 16
            sl = pl.ds(al, 16)
            eloc[sl] = jnp.where(al + lane == evs, zi + (t + 1), eloc[sl])
            return (hidx2, hsc3)

        lax.fori_loop(cache, s - 1, step, (hidx0, hsc0))
        pltpu.sync_copy(eloc, e_hbm.at[h])


def _attnv_body(a_ref, e_ref, v_ref, o_ref, *, s, h, rb):
    hh = pl.program_id(0)
    rbi = pl.program_id(1)
    a = a_ref[0]
    e_full = e_ref[...]
    hrow = jax.lax.broadcasted_iota(jnp.int32, (h, s), 0)
    e_h = jnp.max(jnp.where(hrow == hh, e_full, 0), axis=0, keepdims=True)
    row = rbi * rb + jax.lax.broadcasted_iota(jnp.int32, (rb, s), 0)
    col = jax.lax.broadcasted_iota(jnp.int32, (rb, s), 1)
    msk = (col <= row) & (row < e_h)
    aa = jnp.where(msk, a, NEG)
    m = jnp.max(aa, axis=1, keepdims=True)
    p = jnp.exp(aa - m)
    p = p / jnp.sum(p, axis=1, keepdims=True)
    dn = (((1,), (0,)), ((), ()))
    o_ref[0] = jax.lax.dot_general(p, v_ref[0], dn, preferred_element_type=jnp.float32)


def _outproj_body(o_ref, wot_ref, y_ref):
    hh = pl.program_id(1)

    @pl.when(hh == 0)
    def _init():
        y_ref[...] = jnp.zeros_like(y_ref)

    dn = (((1,), (0,)), ((), ()))
    y_ref[...] += jax.lax.dot_general(o_ref[0], wot_ref[0], dn, preferred_element_type=jnp.float32)


def _run(hs, Wq, Wk, Wv, Wo, *, s, hid, nheads, d, interpret=False):
    heavy = int(0.1 * s)
    recent = int(0.1 * s)
    cache = heavy + recent
    scale = 1.0 / float(np.sqrt(d).astype(np.float32))
    rb = min(256, s)
    nrb = s // rb
    rb3 = 8
    f32 = jnp.float32

    inv_freq = 1.0 / (10000.0 ** (jnp.arange(0, d, 2, dtype=f32) / d))
    t_ar = jnp.arange(s, dtype=f32)
    freqs = jnp.einsum('i,j->ij', t_ar, inv_freq)
    emb = jnp.concatenate([freqs, freqs], axis=-1)
    cos, sin = jnp.cos(emb), jnp.sin(emb)

    wq3 = Wq.reshape(nheads, d, hid)
    wk3 = Wk.reshape(nheads, d, hid)
    wv3 = Wv.reshape(nheads, d, hid)
    wot3 = Wo.T.reshape(nheads, d, hid)

    q, k, v = pl.pallas_call(
        _proj_body,
        grid=(nheads, nrb),
        in_specs=[
            pl.BlockSpec((rb, hid), lambda hh, i: (i, 0)),
            pl.BlockSpec((1, d, hid), lambda hh, i: (hh, 0, 0)),
            pl.BlockSpec((1, d, hid), lambda hh, i: (hh, 0, 0)),
            pl.BlockSpec((1, d, hid), lambda hh, i: (hh, 0, 0)),
        ],
        out_specs=[
            pl.BlockSpec((1, rb, d), lambda hh, i: (hh, i, 0)),
            pl.BlockSpec((1, rb, d), lambda hh, i: (hh, i, 0)),
            pl.BlockSpec((1, rb, d), lambda hh, i: (hh, i, 0)),
        ],
        out_shape=[jax.ShapeDtypeStruct((nheads, s, d), f32)] * 3,
        interpret=interpret,
    )(hs, wq3, wk3, wv3)

    a = pl.pallas_call(
        functools.partial(_scores_body, d=d, scale=scale),
        grid=(nheads, nrb),
        in_specs=[
            pl.BlockSpec((1, rb, d), lambda hh, i: (hh, i, 0)),
            pl.BlockSpec((1, s, d), lambda hh, i: (hh, 0, 0)),
            pl.BlockSpec((rb, d), lambda hh, i: (i, 0)),
            pl.BlockSpec((rb, d), lambda hh, i: (i, 0)),
            pl.BlockSpec((s, d), lambda hh, i: (0, 0)),
            pl.BlockSpec((s, d), lambda hh, i: (0, 0)),
        ],
        out_specs=pl.BlockSpec((1, rb, s), lambda hh, i: (hh, i, 0)),
        out_shape=jax.ShapeDtypeStruct((nheads, s, s), f32),
        interpret=interpret,
    )(q, k, cos, sin, cos, sin)

    if not interpret and s >= 2048:
        score0 = jnp.zeros((nheads, s), f32)
        e0 = jnp.zeros((nheads, s), jnp.int32)
        cfl = (cache // rb3) * rb3
        _, score_dense = pl.pallas_call(
            functools.partial(_evict_body, s=s, h=nheads, rb=rb3, w=512,
                              w_prev=0, t0=0, recent=recent, cache=cache,
                              do_evict=False),
            grid=(cfl // rb3,),
            in_specs=[
                pl.BlockSpec((nheads, rb3, 512), lambda tb: (0, tb, 0)),
                pl.BlockSpec((nheads, 512), lambda tb: (0, 0)),
                pl.BlockSpec((nheads, 512), lambda tb: (0, 0)),
            ],
            out_specs=[
                pl.BlockSpec((nheads, 512), lambda tb: (0, 0)),
                pl.BlockSpec((nheads, 512), lambda tb: (0, 0)),
            ],
            out_shape=[jax.ShapeDtypeStruct((nheads, s), jnp.int32),
                       jax.ShapeDtypeStruct((nheads, s), f32)],
            interpret=interpret,
        )(a, score0, e0)
        info = plsc.get_sparse_core_info()
        mesh = plsc.VectorSubcoreMesh(core_axis_name="c", subcore_axis_name="s")
        e = pl.kernel(
            functools.partial(_sc_evict_body, s=s, nheads=nheads, heavy=heavy,
                              recent=recent, cache=cache, nc=info.num_cores),
            mesh=mesh,
            out_type=jax.ShapeDtypeStruct((nheads, s), jnp.int32),
            scratch_types=[
                pltpu.VMEM((4 * 256,), f32),
                pltpu.VMEM((s + 16,), f32),
                pltpu.VMEM((s,), jnp.int32),
                pltpu.VMEM((448,), jnp.int32),
                pltpu.VMEM((384,), jnp.int32),
                pltpu.VMEM((448,), f32),
                pltpu.VMEM((384,), f32),
                pltpu.SemaphoreType.DMA((4,)),
                pltpu.SemaphoreType.DMA((8,)),
            ],
        )(a.reshape(-1), score_dense.reshape(-1))
        return _tail(a, e, v, wot3, s=s, hid=hid, nheads=nheads, d=d,
                     rb=rb, nrb=nrb, f32=f32, interpret=interpret)
    cfl = (cache // rb3) * rb3
    if s >= 2048:
        regions = [(0, cfl, 512, 0, False),
                   (cfl, 512, 512, 512, True),
                   (512, 1024, 1024, 512, True),
                   (1024, 1536, 1536, 1024, True),
                   (1536, s, s, 1536, True)]
    else:
        regions = [(0, cfl, s, 0, False), (cfl, s, s, s, True)]
    score_st = jnp.zeros((nheads, s), f32)
    e = jnp.zeros((nheads, s), jnp.int32)
    for (t0, t1, w, w_prev, do_evict) in regions:
        e, score_st = pl.pallas_call(
            functools.partial(_evict_body, s=s, h=nheads, rb=rb3, w=w,
                              w_prev=w_prev, t0=t0, recent=recent, cache=cache,
                              do_evict=do_evict),
            grid=((t1 - t0) // rb3,),
            in_specs=[
                pl.BlockSpec((nheads, rb3, w), lambda tb, t0b=t0 // rb3: (0, t0b + tb, 0)),
                pl.BlockSpec((nheads, w), lambda tb: (0, 0)),
                pl.BlockSpec((nheads, w), lambda tb: (0, 0)),
            ],
            out_specs=[
                pl.BlockSpec((nheads, w), lambda tb: (0, 0)),
                pl.BlockSpec((nheads, w), lambda tb: (0, 0)),
            ],
            out_shape=[jax.ShapeDtypeStruct((nheads, s), jnp.int32),
                       jax.ShapeDtypeStruct((nheads, s), f32)],
            interpret=interpret,
        )(a, score_st, e)
    return _tail(a, e, v, wot3, s=s, hid=hid, nheads=nheads, d=d,
                 rb=rb, nrb=nrb, f32=f32, interpret=interpret)


def _tail(a, e, v, wot3, *, s, hid, nheads, d, rb, nrb, f32, interpret):
    o = pl.pallas_call(
        functools.partial(_attnv_body, s=s, h=nheads, rb=rb),
        grid=(nheads, nrb),
        in_specs=[
            pl.BlockSpec((1, rb, s), lambda hh, i: (hh, i, 0)),
            pl.BlockSpec((nheads, s), lambda hh, i: (0, 0)),
            pl.BlockSpec((1, s, d), lambda hh, i: (hh, 0, 0)),
        ],
        out_specs=pl.BlockSpec((1, rb, d), lambda hh, i: (hh, i, 0)),
        out_shape=jax.ShapeDtypeStruct((nheads, s, d), f32),
        interpret=interpret,
    )(a, e, v)

    y = pl.pallas_call(
        _outproj_body,
        grid=(nrb, nheads),
        in_specs=[
            pl.BlockSpec((1, rb, d), lambda i, hh: (hh, i, 0)),
            pl.BlockSpec((1, d, hid), lambda i, hh: (hh, 0, 0)),
        ],
        out_specs=pl.BlockSpec((rb, hid), lambda i, hh: (i, 0)),
        out_shape=jax.ShapeDtypeStruct((s, hid), f32),
        interpret=interpret,
    )(o, wot3)
    return y


def kernel(hidden_states, attention_mask, position_ids, Wq, Wk, Wv, Wo):
    b, s, hid = hidden_states.shape
    d = 64
    nheads = hid // d
    y = _run(hidden_states[0], Wq, Wk, Wv, Wo, s=s, hid=hid, nheads=nheads, d=d)
    return y.reshape(b, s, hid)
